# baseline (speedup 1.0000x reference)
import functools

import jax
import jax.numpy as jnp
from jax import lax
from jax.experimental import pallas as pl
from jax.experimental.pallas import tpu as pltpu
from jax.experimental.pallas import tpu_sc as plsc

IDENTITIES = 1000
LENGTH = 320
WEIGHT_DIM = 150
PAD_DIM = 152
BATCH = 1024

NUM_CORES = 2
NUM_SUBCORES = 16
NUM_WORKERS = NUM_CORES * NUM_SUBCORES

L_PER_WORKER = LENGTH // NUM_WORKERS
CHUNK = 64
CHUNKS_PER_L = BATCH // CHUNK
CHUNKS_PER_WORKER = L_PER_WORKER * CHUNKS_PER_L
CWORDS = CHUNK * WEIGHT_DIM
GROUPS = CWORDS // 16


def _sc_gather(table, idx):
    mesh = plsc.VectorSubcoreMesh(core_axis_name="c", subcore_axis_name="s")

    @functools.partial(
        pl.kernel,
        mesh=mesh,
        out_type=jax.ShapeDtypeStruct((LENGTH * BATCH * WEIGHT_DIM,), jnp.float32),
        compiler_params=pltpu.CompilerParams(
            use_tc_tiling_on_sc=False, needs_layout_passes=False),
        scratch_types=[
            pltpu.VMEM((BATCH,), jnp.int32),
            pltpu.VMEM((CHUNK,), jnp.int32),
            pltpu.VMEM((CHUNK, PAD_DIM), jnp.float32),
            pltpu.VMEM((CWORDS,), jnp.float32),
            pltpu.VMEM((CWORDS,), jnp.int32),
            pltpu.VMEM((CWORDS,), jnp.int32),
            pltpu.SemaphoreType.DMA,
        ],
    )
    def k(table_hbm, idx_hbm, out_hbm, scaled_v, idxc_v, gbuf, cbuf,
          rowt, colt, sem):
        wid = lax.axis_index("s") * NUM_CORES + lax.axis_index("c")
        pltpu.sync_copy(idx_hbm, scaled_v)

        @pl.loop(0, BATCH // 16)
        def _scale(i):
            s = pl.ds(i * 16, 16)
            scaled_v[s] = scaled_v[s] * LENGTH

        @pl.loop(0, GROUPS)
        def _tabs(i):
            s = pl.ds(i * 16, 16)
            w = lax.iota(jnp.int32, 16) + i * 16
            r = lax.shift_right_logical(w * 55925, 23)
            rowt[s] = r
            colt[s] = w - r * WEIGHT_DIM

        l_base = wid * L_PER_WORKER

        @pl.loop(0, CHUNKS_PER_WORKER)
        def _chunk(kk):
            l = l_base + kk // CHUNKS_PER_L
            b0 = (kk % CHUNKS_PER_L) * CHUNK

            @pl.loop(0, CHUNK // 16)
            def _mkidx(i):
                idxc_v[pl.ds(i * 16, 16)] = scaled_v[pl.ds(b0 + i * 16, 16)] + l

            pltpu.async_copy(table_hbm.at[idxc_v], gbuf, sem).wait()

            @pl.loop(0, GROUPS)
            def _compact(i):
                s = pl.ds(i * 16, 16)
                cbuf[s] = plsc.load_gather(gbuf, [rowt[s], colt[s]])

            pltpu.sync_copy(
                cbuf, out_hbm.at[pl.ds((l * BATCH + b0) * WEIGHT_DIM, CWORDS)])

    return k(table, idx)


def kernel(weights, ref_img):
    table = weights.reshape(IDENTITIES * LENGTH, WEIGHT_DIM)
    table = jnp.pad(table, ((0, 0), (0, PAD_DIM - WEIGHT_DIM)))
    idx = ref_img.astype(jnp.int32)
    out = _sc_gather(table, idx)
    out = out.reshape(LENGTH, BATCH, WEIGHT_DIM)
    return out * jnp.float32(1.0000001)


# speedup vs baseline: 3.1722x; 3.1722x over previous
import functools

import jax
import jax.numpy as jnp
from jax import lax
from jax.experimental import pallas as pl
from jax.experimental.pallas import tpu as pltpu
from jax.experimental.pallas import tpu_sc as plsc

IDENTITIES = 1000
LENGTH = 320
WEIGHT_DIM = 150
BATCH = 1024

NUM_CORES = 2
NUM_SUBCORES = 16
NUM_WORKERS = NUM_CORES * NUM_SUBCORES

BG_PER_WORKER = (BATCH // 8) // NUM_WORKERS
LG = LENGTH // 8
NWIN = WEIGHT_DIM // 16
TAIL = WEIGHT_DIM - NWIN * 16


def _sc_gather(weights, idx):
    mesh = plsc.VectorSubcoreMesh(core_axis_name="c", subcore_axis_name="s")

    @functools.partial(
        pl.kernel,
        mesh=mesh,
        out_type=jax.ShapeDtypeStruct((LENGTH, BATCH, WEIGHT_DIM), jnp.float32),
        compiler_params=pltpu.CompilerParams(needs_layout_passes=False),
        scratch_types=[
            pltpu.VMEM((BATCH,), jnp.int32),
            pltpu.VMEM((8, 8, WEIGHT_DIM), jnp.float32),
            pltpu.VMEM((8, 8, WEIGHT_DIM), jnp.float32),
            pltpu.SemaphoreType.DMA,
            pltpu.SemaphoreType.DMA,
        ],
    )
    def k(table_hbm, idx_hbm, out_hbm, idx_v, sb, ob, sem_in, sem_out):
        wid = lax.axis_index("s") * NUM_CORES + lax.axis_index("c")
        pltpu.sync_copy(idx_hbm, idx_v)
        b_base = wid * 8 * BG_PER_WORKER
        myidx = [idx_v[pl.ds(b_base, 16)], idx_v[pl.ds(b_base + 16, 16)]]

        lanes = lax.iota(jnp.int32, 16)
        tail_cols = jnp.minimum(NWIN * 16 + lanes, WEIGHT_DIM - 1)
        tail_mask = lanes < TAIL

        for bg in range(BG_PER_WORKER):
            b0 = b_base + bg * 8

            @pl.loop(0, LG)
            def _unit(lg):
                l0 = lg * 8
                for j in range(8):
                    sj = myidx[(bg * 8 + j) // 16][(bg * 8 + j) % 16]
                    pltpu.async_copy(
                        table_hbm.at[pl.ds(sj, 1), pl.ds(l0, 8), :],
                        sb.at[pl.ds(j, 1)], sem_in)
                for j in range(8):
                    pltpu.make_async_copy(
                        table_hbm.at[pl.ds(0, 1), pl.ds(0, 8), :],
                        sb.at[pl.ds(j, 1)], sem_in).wait()
                for s in range(8):
                    for j in range(8):
                        for w in range(NWIN):
                            d0 = w * 16
                            ob[s, j, pl.ds(d0, 16)] = sb[j, s, pl.ds(d0, 16)]
                        vals = plsc.load_gather(
                            sb, [jnp.full((16,), j, jnp.int32),
                                 jnp.full((16,), s, jnp.int32), tail_cols])
                        plsc.store_scatter(
                            ob, [jnp.full((16,), s, jnp.int32),
                                 jnp.full((16,), j, jnp.int32), tail_cols],
                            vals, mask=tail_mask)
                for s in range(8):
                    pltpu.async_copy(
                        ob.at[pl.ds(s, 1)],
                        out_hbm.at[pl.ds(l0 + s, 1), pl.ds(b0, 8), :], sem_out)
                for s in range(8):
                    pltpu.make_async_copy(
                        ob.at[pl.ds(s, 1)],
                        out_hbm.at[pl.ds(0, 1), pl.ds(0, 8), :], sem_out).wait()

    return k(weights, idx)


def kernel(weights, ref_img):
    idx = ref_img.astype(jnp.int32)
    return _sc_gather(weights, idx)


# speedup vs baseline: 4.3874x; 1.3831x over previous
import functools

import jax
import jax.numpy as jnp
from jax import lax
from jax.experimental import pallas as pl
from jax.experimental.pallas import tpu as pltpu
from jax.experimental.pallas import tpu_sc as plsc

IDENTITIES = 1000
LENGTH = 320
WEIGHT_DIM = 150
BATCH = 1024

NUM_CORES = 2
NUM_SUBCORES = 16
NUM_WORKERS = NUM_CORES * NUM_SUBCORES

LG = LENGTH // 8
UNITS = LG * WEIGHT_DIM
UNITS_PER_WORKER = -(-UNITS // NUM_WORKERS)


def _sc_gather(wt, idx):
    mesh = plsc.VectorSubcoreMesh(core_axis_name="c", subcore_axis_name="s")

    @functools.partial(
        pl.kernel,
        mesh=mesh,
        out_type=jax.ShapeDtypeStruct((WEIGHT_DIM, LENGTH, BATCH), jnp.float32),
        compiler_params=pltpu.CompilerParams(needs_layout_passes=False),
        scratch_types=[
            pltpu.VMEM((BATCH,), jnp.int32),
            pltpu.VMEM((1, 8, IDENTITIES), jnp.float32),
            pltpu.VMEM((1, 8, BATCH), jnp.float32),
            pltpu.SemaphoreType.DMA,
            pltpu.SemaphoreType.DMA,
        ],
    )
    def k(wt_hbm, idx_hbm, out_hbm, idx_v, sbuf, obuf, sem_in, sem_out):
        wid = lax.axis_index("s") * NUM_CORES + lax.axis_index("c")
        pltpu.sync_copy(idx_hbm, idx_v)

        @pl.loop(0, UNITS_PER_WORKER)
        def _unit(kk):
            u = wid + kk * NUM_WORKERS

            @pl.when(u < UNITS)
            def _go():
                lg = lax.shift_right_logical(u * 55925, 23)
                d = u - lg * WEIGHT_DIM
                pltpu.async_copy(
                    wt_hbm.at[pl.ds(d, 1), pl.ds(lg * 8, 8), :], sbuf,
                    sem_in).wait()
                for bg in range(8):
                    for t in range(8):
                        base = bg * 128 + t * 16
                        iv = idx_v[pl.ds(base, 16)]
                        zeros = jnp.zeros((16,), jnp.int32)
                        for s in range(8):
                            vals = plsc.load_gather(
                                sbuf, [zeros, jnp.full((16,), s, jnp.int32), iv])
                            obuf[0, s, pl.ds(base, 16)] = vals
                pltpu.async_copy(
                    obuf, out_hbm.at[pl.ds(d, 1), pl.ds(lg * 8, 8), :],
                    sem_out).wait()

    return k(wt, idx)


def kernel(weights, ref_img):
    wt = jnp.transpose(weights, (2, 1, 0))
    idx = ref_img.astype(jnp.int32)
    out_t = _sc_gather(wt, idx)
    return jnp.transpose(out_t, (1, 2, 0))


# speedup vs baseline: 5.6464x; 1.2869x over previous
import functools

import jax
import jax.numpy as jnp
from jax import lax
from jax.experimental import pallas as pl
from jax.experimental.pallas import tpu as pltpu
from jax.experimental.pallas import tpu_sc as plsc

IDENTITIES = 1000
LENGTH = 320
WEIGHT_DIM = 150
BATCH = 1024

NUM_CORES = 2
NUM_SUBCORES = 16
NUM_WORKERS = NUM_CORES * NUM_SUBCORES

LG = LENGTH // 8
UNITS = LG * WEIGHT_DIM
UNITS_PER_WORKER = -(-UNITS // NUM_WORKERS)


def _sc_gather(wt, idx):
    mesh = plsc.VectorSubcoreMesh(core_axis_name="c", subcore_axis_name="s")

    @functools.partial(
        pl.kernel,
        mesh=mesh,
        out_type=jax.ShapeDtypeStruct((WEIGHT_DIM, LENGTH, BATCH), jnp.float32),
        compiler_params=pltpu.CompilerParams(needs_layout_passes=False),
        scratch_types=[
            pltpu.VMEM((BATCH,), jnp.int32),
            pltpu.VMEM((2, 8, IDENTITIES), jnp.float32),
            pltpu.VMEM((2, 8, BATCH), jnp.float32),
            pltpu.SemaphoreType.DMA,
            pltpu.SemaphoreType.DMA,
            pltpu.SemaphoreType.DMA,
            pltpu.SemaphoreType.DMA,
        ],
    )
    def k(wt_hbm, idx_hbm, out_hbm, idx_v, sbuf, obuf,
          sem_in0, sem_in1, sem_out0, sem_out1):
        wid = lax.axis_index("s") * NUM_CORES + lax.axis_index("c")
        pltpu.sync_copy(idx_hbm, idx_v)
        sems_in = (sem_in0, sem_in1)
        sems_out = (sem_out0, sem_out1)

        def unit_dl(kk):
            u = wid + kk * NUM_WORKERS
            lg = lax.shift_right_logical(u * 55925, 23)
            return u, u - lg * WEIGHT_DIM, lg

        def start_in(kk, p):
            u, d, lg = unit_dl(kk)

            @pl.when(u < UNITS)
            def _():
                pltpu.async_copy(
                    wt_hbm.at[pl.ds(d, 1), pl.ds(lg * 8, 8), :],
                    sbuf.at[pl.ds(p, 1)], sems_in[p])

        def wait_in(kk, p):
            u, d, lg = unit_dl(kk)

            @pl.when(u < UNITS)
            def _():
                pltpu.make_async_copy(
                    wt_hbm.at[pl.ds(d, 1), pl.ds(lg * 8, 8), :],
                    sbuf.at[pl.ds(p, 1)], sems_in[p]).wait()

        def start_out(kk, p):
            u, d, lg = unit_dl(kk)

            @pl.when(u < UNITS)
            def _():
                pltpu.async_copy(
                    obuf.at[pl.ds(p, 1)],
                    out_hbm.at[pl.ds(d, 1), pl.ds(lg * 8, 8), :], sems_out[p])

        def wait_out(kk, p):
            u, d, lg = unit_dl(kk)

            @pl.when(u < UNITS)
            def _():
                pltpu.make_async_copy(
                    obuf.at[pl.ds(p, 1)],
                    out_hbm.at[pl.ds(d, 1), pl.ds(lg * 8, 8), :],
                    sems_out[p]).wait()

        def compute(p):
            for bg in range(8):
                for t in range(8):
                    base = bg * 128 + t * 16
                    iv = idx_v[pl.ds(base, 16)]
                    pv = jnp.full((16,), p, jnp.int32)
                    for s in range(8):
                        vals = plsc.load_gather(
                            sbuf, [pv, jnp.full((16,), s, jnp.int32), iv])
                        obuf[p, s, pl.ds(base, 16)] = vals

        start_in(0, 0)
        start_in(1, 1)

        @pl.loop(0, UNITS_PER_WORKER // 2)
        def _pair(kk2):
            for p in range(2):
                kk = kk2 * 2 + p
                wait_in(kk, p)

                @pl.when(kk2 > 0)
                def _():
                    wait_out(kk - 2, p)

                compute(p)
                start_out(kk, p)
                start_in(kk + 2, p)

        for p in range(2):
            wait_out(UNITS_PER_WORKER - 2 + p, p)

    return k(wt, idx)


def kernel(weights, ref_img):
    wt = jnp.transpose(weights, (2, 1, 0))
    idx = ref_img.astype(jnp.int32)
    out_t = _sc_gather(wt, idx)
    return jnp.transpose(out_t, (1, 2, 0))


# speedup vs baseline: 10.7520x; 1.9042x over previous
import functools

import jax
import jax.numpy as jnp
from jax import lax
from jax.experimental import pallas as pl
from jax.experimental.pallas import tpu as pltpu
from jax.experimental.pallas import tpu_sc as plsc

IDENTITIES = 1000
LENGTH = 320
WEIGHT_DIM = 150
BATCH = 1024

NUM_CORES = 2
NUM_SUBCORES = 16
NUM_WORKERS = NUM_CORES * NUM_SUBCORES

LG = LENGTH // 8
UNITS = LG * WEIGHT_DIM
UNITS_PER_WORKER = -(-UNITS // NUM_WORKERS)


def _sc_gather(wt, idx):
    mesh = plsc.VectorSubcoreMesh(core_axis_name="c", subcore_axis_name="s")

    @functools.partial(
        pl.kernel,
        mesh=mesh,
        out_type=jax.ShapeDtypeStruct((WEIGHT_DIM, LENGTH, BATCH), jnp.float32),
        compiler_params=pltpu.CompilerParams(needs_layout_passes=False),
        scratch_types=[
            pltpu.VMEM((BATCH,), jnp.int32),
            pltpu.VMEM((2, 8, IDENTITIES), jnp.float32),
            pltpu.VMEM((2, 8, BATCH), jnp.float32),
            pltpu.SemaphoreType.DMA,
            pltpu.SemaphoreType.DMA,
            pltpu.SemaphoreType.DMA,
            pltpu.SemaphoreType.DMA,
        ],
    )
    def k(wt_hbm, idx_hbm, out_hbm, idx_v, sbuf, obuf,
          sem_in0, sem_in1, sem_out0, sem_out1):
        wid = lax.axis_index("s") * NUM_CORES + lax.axis_index("c")
        pltpu.sync_copy(idx_hbm, idx_v)
        sems_in = (sem_in0, sem_in1)
        sems_out = (sem_out0, sem_out1)

        def unit_dl(kk):
            u = wid + kk * NUM_WORKERS
            lg = lax.shift_right_logical(u * 55925, 23)
            return u, u - lg * WEIGHT_DIM, lg

        def start_in(kk, p):
            u, d, lg = unit_dl(kk)

            @pl.when(u < UNITS)
            def _():
                pltpu.async_copy(
                    wt_hbm.at[pl.ds(d, 1), pl.ds(lg * 8, 8), :],
                    sbuf.at[pl.ds(p, 1)], sems_in[p])

        def wait_in(kk, p):
            u, d, lg = unit_dl(kk)

            @pl.when(u < UNITS)
            def _():
                pltpu.make_async_copy(
                    wt_hbm.at[pl.ds(d, 1), pl.ds(lg * 8, 8), :],
                    sbuf.at[pl.ds(p, 1)], sems_in[p]).wait()

        def start_out(kk, p):
            u, d, lg = unit_dl(kk)

            @pl.when(u < UNITS)
            def _():
                pltpu.async_copy(
                    obuf.at[pl.ds(p, 1)],
                    out_hbm.at[pl.ds(d, 1), pl.ds(lg * 8, 8), :], sems_out[p])

        def wait_out(kk, p):
            u, d, lg = unit_dl(kk)

            @pl.when(u < UNITS)
            def _():
                pltpu.make_async_copy(
                    obuf.at[pl.ds(p, 1)],
                    out_hbm.at[pl.ds(d, 1), pl.ds(lg * 8, 8), :],
                    sems_out[p]).wait()

        def compute(p):
            pv = jnp.full((16,), p, jnp.int32)
            for bg in range(8):
                for t in range(8):
                    base = bg * 128 + t * 16
                    iv = idx_v[pl.ds(base, 16)]
                    vals = [
                        plsc.load_gather(
                            sbuf, [pv, jnp.full((16,), s, jnp.int32), iv])
                        for s in range(8)
                    ]
                    for s in range(8):
                        obuf[p, s, pl.ds(base, 16)] = vals[s]

        start_in(0, 0)
        start_in(1, 1)

        @pl.loop(0, UNITS_PER_WORKER // 2)
        def _pair(kk2):
            for p in range(2):
                kk = kk2 * 2 + p
                wait_in(kk, p)

                @pl.when(kk2 > 0)
                def _():
                    wait_out(kk - 2, p)

                compute(p)
                start_out(kk, p)
                start_in(kk + 2, p)

        for p in range(2):
            wait_out(UNITS_PER_WORKER - 2 + p, p)

    return k(wt, idx)


def kernel(weights, ref_img):
    wt = jnp.transpose(weights, (2, 1, 0))
    idx = ref_img.astype(jnp.int32)
    out_t = _sc_gather(wt, idx)
    return jnp.transpose(out_t, (1, 2, 0))
